# baseline (speedup 1.0000x reference)
import functools

import jax
import jax.numpy as jnp
from jax import lax
from jax.experimental import pallas as pl
from jax.experimental.pallas import tpu as pltpu
from jax.experimental.pallas import tpu_sc as plsc

_DIM = 256
_K = 8192
_EPS = 1e-05
_NB = 128
_CHUNK = 2736
_NEG_INF = float("-inf")


def _vq_tc_body(a_ref, eb_ref, xx_ref, ee_ref, csize_ref,
                idx_ref, loss_ref, perp_ref):
    i = pl.program_id(0)

    @pl.when(i == 0)
    def _init():
        p = csize_ref[...]
        perp_ref[0, 0] = jnp.exp(-jnp.sum(p * jnp.log(p + _EPS)))
        loss_ref[0, 0] = 0.0

    xe2t = lax.dot_general(eb_ref[...], a_ref[...], (((1,), (1,)), ((), ())),
                           preferred_element_type=jnp.float32)
    dist_t = -((xx_ref[0, 0, :][None, :] - xe2t) + ee_ref[...])

    riota = lax.broadcasted_iota(jnp.int32, (128, _NB), 0)
    ninf = jnp.full((128, _NB), _NEG_INF, jnp.float32)

    n_ch = (_K + _CHUNK - 1) // _CHUNK
    m = [ninf for _ in range(n_ch)]
    ix = [jnp.zeros((128, _NB), jnp.int32) for _ in range(n_ch)]
    for c in range(_K // 128):
        t = dist_t[c * 128:(c + 1) * 128, :]
        code = riota + (c * 128)
        lo, hi = c * 128, (c + 1) * 128
        for ch in range(n_ch):
            s, e = ch * _CHUNK, min((ch + 1) * _CHUNK, _K)
            if hi <= s or lo >= e:
                continue
            if s <= lo and hi <= e:
                tc = t
            else:
                tc = jnp.where((code >= s) & (code < e), t, ninf)
            take = tc > m[ch]
            m[ch] = jnp.where(take, tc, m[ch])
            ix[ch] = jnp.where(take, code, ix[ch])

    for ch in range(n_ch):
        mv, iv = m[ch], ix[ch]
        s = 64
        while s >= 1:
            a_v, b_v = mv[:s, :], mv[s:2 * s, :]
            a_i, b_i = iv[:s, :], iv[s:2 * s, :]
            take_b = (b_v > a_v) | ((b_v == a_v) & (b_i < a_i))
            mv = jnp.where(take_b, b_v, a_v)
            iv = jnp.where(take_b, b_i, a_i)
            s //= 2
        m[ch], ix[ch] = mv, iv

    run = m[0].astype(jnp.bfloat16).astype(jnp.float32)
    idx = ix[0]
    for ch in range(1, n_ch):
        take = m[ch] > run
        idx = jnp.where(take, ix[ch], idx)
        run = jnp.where(take, m[ch], run)
        run = run.astype(jnp.bfloat16).astype(jnp.float32)

    idx_ref[0, 0, :] = idx[0, :]
    loss_ref[0, 0] += -jnp.sum(run)


def _vq_distances(a_bf16, embed_bf16, xx3, ee2, csize2d):
    nsteps = a_bf16.shape[0] // _NB
    idx3, loss, perp = pl.pallas_call(
        _vq_tc_body,
        grid=(nsteps,),
        in_specs=[
            pl.BlockSpec((_NB, _DIM), lambda i: (i, 0)),
            pl.BlockSpec((_K, _DIM), lambda i: (0, 0)),
            pl.BlockSpec((1, 1, _NB), lambda i: (i, 0, 0)),
            pl.BlockSpec((_K, 1), lambda i: (0, 0)),
            pl.BlockSpec(csize2d.shape, lambda i: (0, 0)),
        ],
        out_specs=[
            pl.BlockSpec((1, 1, _NB), lambda i: (i, 0, 0)),
            pl.BlockSpec(memory_space=pltpu.SMEM),
            pl.BlockSpec(memory_space=pltpu.SMEM),
        ],
        out_shape=[
            jax.ShapeDtypeStruct((nsteps, 1, _NB), jnp.int32),
            jax.ShapeDtypeStruct((1, 1), jnp.float32),
            jax.ShapeDtypeStruct((1, 1), jnp.float32),
        ],
    )(a_bf16, embed_bf16, xx3, ee2, csize2d)
    return idx3, loss, perp


def _sc_gather(idx_flat, table, n_rows):
    NC, NS = 2, 16
    NW = NC * NS
    b_per_w = n_rows // NW
    CB = 96
    C = b_per_w // CB
    mesh = plsc.VectorSubcoreMesh(core_axis_name="c", subcore_axis_name="s",
                                  num_cores=NC, num_subcores=NS)

    @functools.partial(
        pl.kernel, mesh=mesh,
        out_type=jax.ShapeDtypeStruct((n_rows, _DIM), jnp.float32),
        scratch_types=[
            pltpu.VMEM((b_per_w,), jnp.int32),
            pltpu.VMEM((b_per_w, _DIM), jnp.float32),
            pltpu.SemaphoreType.DMA,
        ],
    )
    def k(idx_hbm, table_hbm, out_hbm, idx_v, rows_v, sem):
        wid = lax.axis_index("s") * NC + lax.axis_index("c")
        base = wid * b_per_w
        pltpu.sync_copy(idx_hbm.at[pl.ds(base, b_per_w)], idx_v)
        copies = [
            pltpu.async_copy(table_hbm.at[idx_v.at[pl.ds(c * CB, CB)]],
                             rows_v.at[pl.ds(c * CB, CB), :], sem)
            for c in range(C)
        ]
        for cp in copies:
            cp.wait()
        pltpu.sync_copy(rows_v, out_hbm.at[pl.ds(base, b_per_w)])

    return k(idx_flat, table)


def kernel(x, embed, cluster_size):
    x = x.astype(jnp.float32)
    shape = x.shape
    n = shape[0] * shape[1]
    x_flat = x.reshape(n, _DIM)
    a_bf16 = (2.0 * x_flat).astype(jnp.bfloat16)
    embed_bf16 = embed.astype(jnp.bfloat16)
    xx3 = jnp.sum(x_flat * x_flat, axis=-1).reshape(n // _NB, 1, _NB)
    ee2 = jnp.sum(embed * embed, axis=-1).reshape(_K, 1)
    csize2d = cluster_size.reshape(_K // 128, 128)

    idx3, loss, perp = _vq_distances(a_bf16, embed_bf16, xx3, ee2, csize2d)
    idx_flat = idx3.reshape(n)
    quantize = _sc_gather(idx_flat, embed, n)

    embed_ind = idx_flat.reshape(shape[:-1])
    quantize_st = quantize.reshape(shape)
    commit_loss = loss[0, 0] / jnp.float32(n * _DIM)
    perplexity = perp[0, 0]
    return (quantize_st, embed_ind, commit_loss, perplexity)


# speedup vs baseline: 1.1746x; 1.1746x over previous
import functools

import jax
import jax.numpy as jnp
from jax import lax
from jax.experimental import pallas as pl
from jax.experimental.pallas import tpu as pltpu
from jax.experimental.pallas import tpu_sc as plsc

_DIM = 256
_K = 8192
_EPS = 1e-05
_NB = 256
_CHUNK = 2736
_NEG_INF = float("-inf")


def _vq_tc_body(a_ref, eb_ref, xx_ref, ee_ref, csize_ref,
                idx_ref, loss_ref, perp_ref):
    i = pl.program_id(0)

    @pl.when(i == 0)
    def _init():
        p = csize_ref[...]
        perp_ref[0, 0] = jnp.exp(-jnp.sum(p * jnp.log(p + _EPS)))
        loss_ref[0, 0] = 0.0

    xe2t = lax.dot_general(eb_ref[...], a_ref[...], (((1,), (1,)), ((), ())),
                           preferred_element_type=jnp.float32)
    dist_t = -((xx_ref[0, 0, :][None, :] - xe2t) + ee_ref[...])

    riota = lax.broadcasted_iota(jnp.int32, (128, _NB), 0)
    ninf = jnp.full((128, _NB), _NEG_INF, jnp.float32)

    n_ch = (_K + _CHUNK - 1) // _CHUNK
    m = [ninf for _ in range(n_ch)]
    ix = [jnp.zeros((128, _NB), jnp.int32) for _ in range(n_ch)]
    for c in range(_K // 128):
        t = dist_t[c * 128:(c + 1) * 128, :]
        code = riota + (c * 128)
        lo, hi = c * 128, (c + 1) * 128
        for ch in range(n_ch):
            s, e = ch * _CHUNK, min((ch + 1) * _CHUNK, _K)
            if hi <= s or lo >= e:
                continue
            if s <= lo and hi <= e:
                tc = t
            else:
                tc = jnp.where((code >= s) & (code < e), t, ninf)
            take = tc > m[ch]
            m[ch] = jnp.where(take, tc, m[ch])
            ix[ch] = jnp.where(take, code, ix[ch])

    for ch in range(n_ch):
        mv, iv = m[ch], ix[ch]
        s = 64
        while s >= 1:
            a_v, b_v = mv[:s, :], mv[s:2 * s, :]
            a_i, b_i = iv[:s, :], iv[s:2 * s, :]
            take_b = (b_v > a_v) | ((b_v == a_v) & (b_i < a_i))
            mv = jnp.where(take_b, b_v, a_v)
            iv = jnp.where(take_b, b_i, a_i)
            s //= 2
        m[ch], ix[ch] = mv, iv

    run = m[0].astype(jnp.bfloat16).astype(jnp.float32)
    idx = ix[0]
    for ch in range(1, n_ch):
        take = m[ch] > run
        idx = jnp.where(take, ix[ch], idx)
        run = jnp.where(take, m[ch], run)
        run = run.astype(jnp.bfloat16).astype(jnp.float32)

    idx_ref[0, 0, :] = idx[0, :]
    loss_ref[0, 0] += -jnp.sum(run)


def _vq_distances(a_bf16, embed_bf16, xx3, ee2, csize2d):
    nsteps = a_bf16.shape[0] // _NB
    idx3, loss, perp = pl.pallas_call(
        _vq_tc_body,
        grid=(nsteps,),
        in_specs=[
            pl.BlockSpec((_NB, _DIM), lambda i: (i, 0)),
            pl.BlockSpec((_K, _DIM), lambda i: (0, 0)),
            pl.BlockSpec((1, 1, _NB), lambda i: (i, 0, 0)),
            pl.BlockSpec((_K, 1), lambda i: (0, 0)),
            pl.BlockSpec(csize2d.shape, lambda i: (0, 0)),
        ],
        out_specs=[
            pl.BlockSpec((1, 1, _NB), lambda i: (i, 0, 0)),
            pl.BlockSpec(memory_space=pltpu.SMEM),
            pl.BlockSpec(memory_space=pltpu.SMEM),
        ],
        out_shape=[
            jax.ShapeDtypeStruct((nsteps, 1, _NB), jnp.int32),
            jax.ShapeDtypeStruct((1, 1), jnp.float32),
            jax.ShapeDtypeStruct((1, 1), jnp.float32),
        ],
    )(a_bf16, embed_bf16, xx3, ee2, csize2d)
    return idx3, loss, perp


def _sc_gather(idx_flat, table, n_rows):
    NC, NS = 2, 16
    NW = NC * NS
    b_per_w = n_rows // NW
    CB = 96
    C = b_per_w // CB
    mesh = plsc.VectorSubcoreMesh(core_axis_name="c", subcore_axis_name="s",
                                  num_cores=NC, num_subcores=NS)

    @functools.partial(
        pl.kernel, mesh=mesh,
        out_type=jax.ShapeDtypeStruct((n_rows, _DIM), jnp.float32),
        scratch_types=[
            pltpu.VMEM((b_per_w,), jnp.int32),
            pltpu.VMEM((b_per_w, _DIM), jnp.float32),
            pltpu.SemaphoreType.DMA,
        ],
    )
    def k(idx_hbm, table_hbm, out_hbm, idx_v, rows_v, sem):
        wid = lax.axis_index("s") * NC + lax.axis_index("c")
        base = wid * b_per_w
        pltpu.sync_copy(idx_hbm.at[pl.ds(base, b_per_w)], idx_v)
        copies = [
            pltpu.async_copy(table_hbm.at[idx_v.at[pl.ds(c * CB, CB)]],
                             rows_v.at[pl.ds(c * CB, CB), :], sem)
            for c in range(C)
        ]
        for cp in copies:
            cp.wait()
        pltpu.sync_copy(rows_v, out_hbm.at[pl.ds(base, b_per_w)])

    return k(idx_flat, table)


def kernel(x, embed, cluster_size):
    x = x.astype(jnp.float32)
    shape = x.shape
    n = shape[0] * shape[1]
    x_flat = x.reshape(n, _DIM)
    a_bf16 = (2.0 * x_flat).astype(jnp.bfloat16)
    embed_bf16 = embed.astype(jnp.bfloat16)
    xx3 = jnp.sum(x_flat * x_flat, axis=-1).reshape(n // _NB, 1, _NB)
    ee2 = jnp.sum(embed * embed, axis=-1).reshape(_K, 1)
    csize2d = cluster_size.reshape(_K // 128, 128)

    idx3, loss, perp = _vq_distances(a_bf16, embed_bf16, xx3, ee2, csize2d)
    idx_flat = idx3.reshape(n)
    quantize = _sc_gather(idx_flat, embed, n)

    embed_ind = idx_flat.reshape(shape[:-1])
    quantize_st = quantize.reshape(shape)
    commit_loss = loss[0, 0] / jnp.float32(n * _DIM)
    perplexity = perp[0, 0]
    return (quantize_st, embed_ind, commit_loss, perplexity)


# speedup vs baseline: 1.2069x; 1.0275x over previous
import functools

import jax
import jax.numpy as jnp
from jax import lax
from jax.experimental import pallas as pl
from jax.experimental.pallas import tpu as pltpu
from jax.experimental.pallas import tpu_sc as plsc

_DIM = 256
_K = 8192
_EPS = 1e-05
_NB = 512
_CHUNK = 2736
_NEG_INF = float("-inf")


def _vq_tc_body(a_ref, eb_ref, xx_ref, ee_ref, csize_ref,
                idx_ref, loss_ref, perp_ref):
    i = pl.program_id(0)

    @pl.when(i == 0)
    def _init():
        p = csize_ref[...]
        perp_ref[0, 0] = jnp.exp(-jnp.sum(p * jnp.log(p + _EPS)))
        loss_ref[0, 0] = 0.0

    xe2t = lax.dot_general(eb_ref[...], a_ref[...], (((1,), (1,)), ((), ())),
                           preferred_element_type=jnp.float32)
    dist_t = -((xx_ref[0, 0, :][None, :] - xe2t) + ee_ref[...])

    riota = lax.broadcasted_iota(jnp.int32, (128, _NB), 0)
    ninf = jnp.full((128, _NB), _NEG_INF, jnp.float32)

    n_ch = (_K + _CHUNK - 1) // _CHUNK
    m = [ninf for _ in range(n_ch)]
    ix = [jnp.zeros((128, _NB), jnp.int32) for _ in range(n_ch)]
    for c in range(_K // 128):
        t = dist_t[c * 128:(c + 1) * 128, :]
        code = riota + (c * 128)
        lo, hi = c * 128, (c + 1) * 128
        for ch in range(n_ch):
            s, e = ch * _CHUNK, min((ch + 1) * _CHUNK, _K)
            if hi <= s or lo >= e:
                continue
            if s <= lo and hi <= e:
                tc = t
            else:
                tc = jnp.where((code >= s) & (code < e), t, ninf)
            take = tc > m[ch]
            m[ch] = jnp.where(take, tc, m[ch])
            ix[ch] = jnp.where(take, code, ix[ch])

    for ch in range(n_ch):
        mv, iv = m[ch], ix[ch]
        s = 64
        while s >= 1:
            a_v, b_v = mv[:s, :], mv[s:2 * s, :]
            a_i, b_i = iv[:s, :], iv[s:2 * s, :]
            take_b = (b_v > a_v) | ((b_v == a_v) & (b_i < a_i))
            mv = jnp.where(take_b, b_v, a_v)
            iv = jnp.where(take_b, b_i, a_i)
            s //= 2
        m[ch], ix[ch] = mv, iv

    run = m[0].astype(jnp.bfloat16).astype(jnp.float32)
    idx = ix[0]
    for ch in range(1, n_ch):
        take = m[ch] > run
        idx = jnp.where(take, ix[ch], idx)
        run = jnp.where(take, m[ch], run)
        run = run.astype(jnp.bfloat16).astype(jnp.float32)

    idx_ref[0, 0, :] = idx[0, :]
    loss_ref[0, 0] += -jnp.sum(run)


def _vq_distances(a_bf16, embed_bf16, xx3, ee2, csize2d):
    nsteps = a_bf16.shape[0] // _NB
    idx3, loss, perp = pl.pallas_call(
        _vq_tc_body,
        grid=(nsteps,),
        in_specs=[
            pl.BlockSpec((_NB, _DIM), lambda i: (i, 0)),
            pl.BlockSpec((_K, _DIM), lambda i: (0, 0)),
            pl.BlockSpec((1, 1, _NB), lambda i: (i, 0, 0)),
            pl.BlockSpec((_K, 1), lambda i: (0, 0)),
            pl.BlockSpec(csize2d.shape, lambda i: (0, 0)),
        ],
        out_specs=[
            pl.BlockSpec((1, 1, _NB), lambda i: (i, 0, 0)),
            pl.BlockSpec(memory_space=pltpu.SMEM),
            pl.BlockSpec(memory_space=pltpu.SMEM),
        ],
        out_shape=[
            jax.ShapeDtypeStruct((nsteps, 1, _NB), jnp.int32),
            jax.ShapeDtypeStruct((1, 1), jnp.float32),
            jax.ShapeDtypeStruct((1, 1), jnp.float32),
        ],
    )(a_bf16, embed_bf16, xx3, ee2, csize2d)
    return idx3, loss, perp


def _sc_gather(idx_flat, table, n_rows):
    NC, NS = 2, 16
    NW = NC * NS
    b_per_w = n_rows // NW
    CB = 96
    C = b_per_w // CB
    mesh = plsc.VectorSubcoreMesh(core_axis_name="c", subcore_axis_name="s",
                                  num_cores=NC, num_subcores=NS)

    @functools.partial(
        pl.kernel, mesh=mesh,
        out_type=jax.ShapeDtypeStruct((n_rows, _DIM), jnp.float32),
        scratch_types=[
            pltpu.VMEM((b_per_w,), jnp.int32),
            pltpu.VMEM((b_per_w, _DIM), jnp.float32),
            pltpu.SemaphoreType.DMA,
        ],
    )
    def k(idx_hbm, table_hbm, out_hbm, idx_v, rows_v, sem):
        wid = lax.axis_index("s") * NC + lax.axis_index("c")
        base = wid * b_per_w
        pltpu.sync_copy(idx_hbm.at[pl.ds(base, b_per_w)], idx_v)
        copies = [
            pltpu.async_copy(table_hbm.at[idx_v.at[pl.ds(c * CB, CB)]],
                             rows_v.at[pl.ds(c * CB, CB), :], sem)
            for c in range(C)
        ]
        for cp in copies:
            cp.wait()
        pltpu.sync_copy(rows_v, out_hbm.at[pl.ds(base, b_per_w)])

    return k(idx_flat, table)


def kernel(x, embed, cluster_size):
    x = x.astype(jnp.float32)
    shape = x.shape
    n = shape[0] * shape[1]
    x_flat = x.reshape(n, _DIM)
    a_bf16 = (2.0 * x_flat).astype(jnp.bfloat16)
    embed_bf16 = embed.astype(jnp.bfloat16)
    xx3 = jnp.sum(x_flat * x_flat, axis=-1).reshape(n // _NB, 1, _NB)
    ee2 = jnp.sum(embed * embed, axis=-1).reshape(_K, 1)
    csize2d = cluster_size.reshape(_K // 128, 128)

    idx3, loss, perp = _vq_distances(a_bf16, embed_bf16, xx3, ee2, csize2d)
    idx_flat = idx3.reshape(n)
    quantize = _sc_gather(idx_flat, embed, n)

    embed_ind = idx_flat.reshape(shape[:-1])
    quantize_st = quantize.reshape(shape)
    commit_loss = loss[0, 0] / jnp.float32(n * _DIM)
    perplexity = perp[0, 0]
    return (quantize_st, embed_ind, commit_loss, perplexity)


# speedup vs baseline: 1.2510x; 1.0365x over previous
import functools

import jax
import jax.numpy as jnp
from jax import lax
from jax.experimental import pallas as pl
from jax.experimental.pallas import tpu as pltpu
from jax.experimental.pallas import tpu_sc as plsc

_DIM = 256
_K = 8192
_EPS = 1e-05
_NB = 768
_CHUNK = 2736
_NEG_INF = float("-inf")


def _vq_tc_body(a_ref, eb_ref, xx_ref, ee_ref, csize_ref,
                idx_ref, loss_ref, perp_ref):
    i = pl.program_id(0)

    @pl.when(i == 0)
    def _init():
        p = csize_ref[...]
        perp_ref[0, 0] = jnp.exp(-jnp.sum(p * jnp.log(p + _EPS)))
        loss_ref[0, 0] = 0.0

    xe2t = lax.dot_general(eb_ref[...], a_ref[...], (((1,), (1,)), ((), ())),
                           preferred_element_type=jnp.float32)
    dist_t = -((xx_ref[0, 0, :][None, :] - xe2t) + ee_ref[...])

    riota = lax.broadcasted_iota(jnp.int32, (128, _NB), 0)
    ninf = jnp.full((128, _NB), _NEG_INF, jnp.float32)

    n_ch = (_K + _CHUNK - 1) // _CHUNK
    m = [ninf for _ in range(n_ch)]
    ix = [jnp.zeros((128, _NB), jnp.int32) for _ in range(n_ch)]
    for c in range(_K // 128):
        t = dist_t[c * 128:(c + 1) * 128, :]
        code = riota + (c * 128)
        lo, hi = c * 128, (c + 1) * 128
        for ch in range(n_ch):
            s, e = ch * _CHUNK, min((ch + 1) * _CHUNK, _K)
            if hi <= s or lo >= e:
                continue
            if s <= lo and hi <= e:
                tc = t
            else:
                tc = jnp.where((code >= s) & (code < e), t, ninf)
            take = tc > m[ch]
            m[ch] = jnp.where(take, tc, m[ch])
            ix[ch] = jnp.where(take, code, ix[ch])

    for ch in range(n_ch):
        mv, iv = m[ch], ix[ch]
        s = 64
        while s >= 1:
            a_v, b_v = mv[:s, :], mv[s:2 * s, :]
            a_i, b_i = iv[:s, :], iv[s:2 * s, :]
            take_b = (b_v > a_v) | ((b_v == a_v) & (b_i < a_i))
            mv = jnp.where(take_b, b_v, a_v)
            iv = jnp.where(take_b, b_i, a_i)
            s //= 2
        m[ch], ix[ch] = mv, iv

    run = m[0].astype(jnp.bfloat16).astype(jnp.float32)
    idx = ix[0]
    for ch in range(1, n_ch):
        take = m[ch] > run
        idx = jnp.where(take, ix[ch], idx)
        run = jnp.where(take, m[ch], run)
        run = run.astype(jnp.bfloat16).astype(jnp.float32)

    idx_ref[0, 0, :] = idx[0, :]
    loss_ref[0, 0] += -jnp.sum(run)


def _vq_distances(a_bf16, embed_bf16, xx3, ee2, csize2d):
    nsteps = a_bf16.shape[0] // _NB
    idx3, loss, perp = pl.pallas_call(
        _vq_tc_body,
        grid=(nsteps,),
        in_specs=[
            pl.BlockSpec((_NB, _DIM), lambda i: (i, 0)),
            pl.BlockSpec((_K, _DIM), lambda i: (0, 0)),
            pl.BlockSpec((1, 1, _NB), lambda i: (i, 0, 0)),
            pl.BlockSpec((_K, 1), lambda i: (0, 0)),
            pl.BlockSpec(csize2d.shape, lambda i: (0, 0)),
        ],
        out_specs=[
            pl.BlockSpec((1, 1, _NB), lambda i: (i, 0, 0)),
            pl.BlockSpec(memory_space=pltpu.SMEM),
            pl.BlockSpec(memory_space=pltpu.SMEM),
        ],
        out_shape=[
            jax.ShapeDtypeStruct((nsteps, 1, _NB), jnp.int32),
            jax.ShapeDtypeStruct((1, 1), jnp.float32),
            jax.ShapeDtypeStruct((1, 1), jnp.float32),
        ],
    )(a_bf16, embed_bf16, xx3, ee2, csize2d)
    return idx3, loss, perp


def _sc_gather(idx_flat, table, n_rows):
    NC, NS = 2, 16
    NW = NC * NS
    b_per_w = n_rows // NW
    CB = 96
    C = b_per_w // CB
    mesh = plsc.VectorSubcoreMesh(core_axis_name="c", subcore_axis_name="s",
                                  num_cores=NC, num_subcores=NS)

    @functools.partial(
        pl.kernel, mesh=mesh,
        out_type=jax.ShapeDtypeStruct((n_rows, _DIM), jnp.float32),
        scratch_types=[
            pltpu.VMEM((b_per_w,), jnp.int32),
            pltpu.VMEM((b_per_w, _DIM), jnp.float32),
            pltpu.SemaphoreType.DMA,
        ],
    )
    def k(idx_hbm, table_hbm, out_hbm, idx_v, rows_v, sem):
        wid = lax.axis_index("s") * NC + lax.axis_index("c")
        base = wid * b_per_w
        pltpu.sync_copy(idx_hbm.at[pl.ds(base, b_per_w)], idx_v)
        copies = [
            pltpu.async_copy(table_hbm.at[idx_v.at[pl.ds(c * CB, CB)]],
                             rows_v.at[pl.ds(c * CB, CB), :], sem)
            for c in range(C)
        ]
        for cp in copies:
            cp.wait()
        pltpu.sync_copy(rows_v, out_hbm.at[pl.ds(base, b_per_w)])

    return k(idx_flat, table)


def kernel(x, embed, cluster_size):
    x = x.astype(jnp.float32)
    shape = x.shape
    n = shape[0] * shape[1]
    x_flat = x.reshape(n, _DIM)
    a_bf16 = (2.0 * x_flat).astype(jnp.bfloat16)
    embed_bf16 = embed.astype(jnp.bfloat16)
    xx3 = jnp.sum(x_flat * x_flat, axis=-1).reshape(n // _NB, 1, _NB)
    ee2 = jnp.sum(embed * embed, axis=-1).reshape(_K, 1)
    csize2d = cluster_size.reshape(_K // 128, 128)

    idx3, loss, perp = _vq_distances(a_bf16, embed_bf16, xx3, ee2, csize2d)
    idx_flat = idx3.reshape(n)
    quantize = _sc_gather(idx_flat, embed, n)

    embed_ind = idx_flat.reshape(shape[:-1])
    quantize_st = quantize.reshape(shape)
    commit_loss = loss[0, 0] / jnp.float32(n * _DIM)
    perplexity = perp[0, 0]
    return (quantize_st, embed_ind, commit_loss, perplexity)
